# baseline (speedup 1.0000x reference)
import jax
import jax.numpy as jnp
from jax.experimental import pallas as pl
from jax.experimental.pallas import tpu as pltpu

_BLOCK_N = 32


def _fused_kernel(x_ref, q_ref, w1_ref, b1_ref, w2_ref, b2_ref,
                  w3_ref, b3_ref, o_ref, buf_ref, sem_ref):
    nb, d = x_ref.shape
    rows = nb * d
    i = pl.program_id(0)
    nsteps = pl.num_programs(0)
    slot = jax.lax.rem(i, 2)

    @pl.when(i >= 2)
    def _():
        pltpu.make_async_copy(
            buf_ref.at[slot],
            o_ref.at[pl.ds((i - 2) * rows, rows), :],
            sem_ref.at[slot],
        ).wait()

    xp = jnp.dot(x_ref[...], q_ref[...], preferred_element_type=jnp.float32)

    i_idx = jax.lax.broadcasted_iota(jnp.int32, (d, d), 0)
    j_idx = jax.lax.broadcasted_iota(jnp.int32, (d, d), 1)
    tri = (j_idx < i_idx).astype(jnp.float32)
    xm = (xp[:, None, :] * tri[None, :, :]).reshape(rows, d)
    xm = xm.astype(jnp.bfloat16)

    h = jnp.dot(xm, w1_ref[...], preferred_element_type=jnp.float32)
    h = jnp.maximum(h + b1_ref[...], 0.0).astype(jnp.bfloat16)
    h = jnp.dot(h, w2_ref[...], preferred_element_type=jnp.float32)
    h = jnp.maximum(h + b2_ref[...], 0.0).astype(jnp.bfloat16)
    buf_ref[slot] = (
        jnp.dot(h, w3_ref[...], preferred_element_type=jnp.float32)
        + b3_ref[...])

    pltpu.make_async_copy(
        buf_ref.at[slot],
        o_ref.at[pl.ds(i * rows, rows), :],
        sem_ref.at[slot],
    ).start()

    @pl.when(jnp.logical_and(i == nsteps - 1, nsteps > 1))
    def _():
        pltpu.make_async_copy(
            buf_ref.at[1 - slot],
            o_ref.at[pl.ds((i - 1) * rows, rows), :],
            sem_ref.at[1 - slot],
        ).wait()

    @pl.when(i == nsteps - 1)
    def _():
        pltpu.make_async_copy(
            buf_ref.at[slot],
            o_ref.at[pl.ds(i * rows, rows), :],
            sem_ref.at[slot],
        ).wait()


@jax.jit
def _forward(x, Q, W1, b1, W2, b2, W3, b3):
    n, d = x.shape
    n_params = W3.shape[1]
    nb = _BLOCK_N

    xb = x.astype(jnp.bfloat16)
    qb = Q.astype(jnp.bfloat16)
    w1b = W1.astype(jnp.bfloat16)
    w2b = W2.astype(jnp.bfloat16)
    w3b = W3.astype(jnp.bfloat16)

    const = lambda i: (0, 0)

    out = pl.pallas_call(
        _fused_kernel,
        grid=(n // nb,),
        in_specs=[
            pl.BlockSpec((nb, d), lambda i: (i, 0)),
            pl.BlockSpec(qb.shape, const),
            pl.BlockSpec(w1b.shape, const),
            pl.BlockSpec(b1.shape, const),
            pl.BlockSpec(w2b.shape, const),
            pl.BlockSpec(b2.shape, const),
            pl.BlockSpec(w3b.shape, const),
            pl.BlockSpec(b3.shape, const),
        ],
        out_specs=pl.BlockSpec(memory_space=pltpu.MemorySpace.HBM),
        out_shape=jax.ShapeDtypeStruct((n * d, n_params), jnp.float32),
        scratch_shapes=[
            pltpu.VMEM((2, nb * d, n_params), jnp.float32),
            pltpu.SemaphoreType.DMA((2,)),
        ],
        compiler_params=pltpu.CompilerParams(
            dimension_semantics=("arbitrary",),
            vmem_limit_bytes=100 * 1024 * 1024,
        ),
    )(xb, qb, w1b, b1, w2b, b2, w3b, b3)

    return out.reshape(n, n_params * d)


def kernel(x, Q, W1, b1, W2, b2, W3, b3):
    return _forward(x, Q, W1, b1, W2, b2, W3, b3)


# speedup vs baseline: 1.0218x; 1.0218x over previous
import jax
import jax.numpy as jnp
from jax.experimental import pallas as pl
from jax.experimental.pallas import tpu as pltpu

_BLOCK_N = 64


def _fused_kernel(x_ref, q_ref, w1_ref, b1_ref, w2_ref, b2_ref,
                  w3_ref, b3_ref, o_ref, buf_ref, sem_ref):
    nb, d = x_ref.shape
    rows = nb * d
    i = pl.program_id(0)
    nsteps = pl.num_programs(0)
    slot = jax.lax.rem(i, 2)

    @pl.when(i >= 2)
    def _():
        pltpu.make_async_copy(
            buf_ref.at[slot],
            o_ref.at[pl.ds((i - 2) * rows, rows), :],
            sem_ref.at[slot],
        ).wait()

    xp = jnp.dot(x_ref[...], q_ref[...], preferred_element_type=jnp.float32)

    i_idx = jax.lax.broadcasted_iota(jnp.int32, (d, d), 0)
    j_idx = jax.lax.broadcasted_iota(jnp.int32, (d, d), 1)
    tri = (j_idx < i_idx).astype(jnp.float32)
    xm = (xp[:, None, :] * tri[None, :, :]).reshape(rows, d)
    xm = xm.astype(jnp.bfloat16)

    h = jnp.dot(xm, w1_ref[...], preferred_element_type=jnp.float32)
    h = jnp.maximum(h + b1_ref[...], 0.0).astype(jnp.bfloat16)
    h = jnp.dot(h, w2_ref[...], preferred_element_type=jnp.float32)
    h = jnp.maximum(h + b2_ref[...], 0.0).astype(jnp.bfloat16)
    buf_ref[slot] = (
        jnp.dot(h, w3_ref[...], preferred_element_type=jnp.float32)
        + b3_ref[...])

    pltpu.make_async_copy(
        buf_ref.at[slot],
        o_ref.at[pl.ds(i * rows, rows), :],
        sem_ref.at[slot],
    ).start()

    @pl.when(jnp.logical_and(i == nsteps - 1, nsteps > 1))
    def _():
        pltpu.make_async_copy(
            buf_ref.at[1 - slot],
            o_ref.at[pl.ds((i - 1) * rows, rows), :],
            sem_ref.at[1 - slot],
        ).wait()

    @pl.when(i == nsteps - 1)
    def _():
        pltpu.make_async_copy(
            buf_ref.at[slot],
            o_ref.at[pl.ds(i * rows, rows), :],
            sem_ref.at[slot],
        ).wait()


@jax.jit
def _forward(x, Q, W1, b1, W2, b2, W3, b3):
    n, d = x.shape
    n_params = W3.shape[1]
    nb = _BLOCK_N

    xb = x.astype(jnp.bfloat16)
    qb = Q.astype(jnp.bfloat16)
    w1b = W1.astype(jnp.bfloat16)
    w2b = W2.astype(jnp.bfloat16)
    w3b = W3.astype(jnp.bfloat16)

    const = lambda i: (0, 0)

    out = pl.pallas_call(
        _fused_kernel,
        grid=(n // nb,),
        in_specs=[
            pl.BlockSpec((nb, d), lambda i: (i, 0)),
            pl.BlockSpec(qb.shape, const),
            pl.BlockSpec(w1b.shape, const),
            pl.BlockSpec(b1.shape, const),
            pl.BlockSpec(w2b.shape, const),
            pl.BlockSpec(b2.shape, const),
            pl.BlockSpec(w3b.shape, const),
            pl.BlockSpec(b3.shape, const),
        ],
        out_specs=pl.BlockSpec(memory_space=pltpu.MemorySpace.HBM),
        out_shape=jax.ShapeDtypeStruct((n * d, n_params), jnp.float32),
        scratch_shapes=[
            pltpu.VMEM((2, nb * d, n_params), jnp.float32),
            pltpu.SemaphoreType.DMA((2,)),
        ],
        compiler_params=pltpu.CompilerParams(
            dimension_semantics=("arbitrary",),
            vmem_limit_bytes=100 * 1024 * 1024,
        ),
    )(xb, qb, w1b, b1, w2b, b2, w3b, b3)

    return out.reshape(n, n_params * d)


def kernel(x, Q, W1, b1, W2, b2, W3, b3):
    return _forward(x, Q, W1, b1, W2, b2, W3, b3)


# speedup vs baseline: 1.5035x; 1.4715x over previous
import jax
import jax.numpy as jnp
from jax.experimental import pallas as pl
from jax.experimental.pallas import tpu as pltpu

_BLOCK_N = 64


def _fused_kernel(x_ref, q_ref, w1_ref, b1_ref, w2_ref, b2_ref,
                  w3_ref, b3_ref, o_ref):
    nb, d = x_ref.shape
    n_params = w3_ref.shape[1]
    rows = nb * d

    xp = jnp.dot(x_ref[...], q_ref[...], preferred_element_type=jnp.float32)

    i_idx = jax.lax.broadcasted_iota(jnp.int32, (d, d), 0)
    j_idx = jax.lax.broadcasted_iota(jnp.int32, (d, d), 1)
    tri = (j_idx < i_idx).astype(jnp.float32)
    xm = (tri[:, None, :] * xp[None, :, :]).reshape(rows, d)
    xm = xm.astype(jnp.bfloat16)

    h = jnp.dot(xm, w1_ref[...], preferred_element_type=jnp.float32)
    h = jnp.maximum(h + b1_ref[...], 0.0).astype(jnp.bfloat16)
    h = jnp.dot(h, w2_ref[...], preferred_element_type=jnp.float32)
    h = jnp.maximum(h + b2_ref[...], 0.0).astype(jnp.bfloat16)
    h = jnp.dot(h, w3_ref[...], preferred_element_type=jnp.float32) + b3_ref[...]

    for i in range(d):
        o_ref[:, i * n_params:(i + 1) * n_params] = h[i * nb:(i + 1) * nb, :]


@jax.jit
def _forward(x, Q, W1, b1, W2, b2, W3, b3):
    n, d = x.shape
    n_params = W3.shape[1]
    nb = _BLOCK_N

    xb = x.astype(jnp.bfloat16)
    qb = Q.astype(jnp.bfloat16)
    w1b = W1.astype(jnp.bfloat16)
    w2b = W2.astype(jnp.bfloat16)
    w3b = W3.astype(jnp.bfloat16)

    const = lambda i: (0, 0)

    out = pl.pallas_call(
        _fused_kernel,
        grid=(n // nb,),
        in_specs=[
            pl.BlockSpec((nb, d), lambda i: (i, 0)),
            pl.BlockSpec(qb.shape, const),
            pl.BlockSpec(w1b.shape, const),
            pl.BlockSpec(b1.shape, const),
            pl.BlockSpec(w2b.shape, const),
            pl.BlockSpec(b2.shape, const),
            pl.BlockSpec(w3b.shape, const),
            pl.BlockSpec(b3.shape, const),
        ],
        out_specs=pl.BlockSpec((nb, d * n_params), lambda i: (i, 0)),
        out_shape=jax.ShapeDtypeStruct((n, d * n_params), jnp.float32),
        compiler_params=pltpu.CompilerParams(
            dimension_semantics=("parallel",)),
    )(xb, qb, w1b, b1, w2b, b2, w3b, b3)

    return out


def kernel(x, Q, W1, b1, W2, b2, W3, b3):
    return _forward(x, Q, W1, b1, W2, b2, W3, b3)


# speedup vs baseline: 1.5292x; 1.0171x over previous
import jax
import jax.numpy as jnp
from jax.experimental import pallas as pl
from jax.experimental.pallas import tpu as pltpu

_BLOCK_N = 128
_I_CHUNK = 32


def _fused_kernel(x_ref, q_ref, w1_ref, b1_ref, w2_ref, b2_ref,
                  w3_ref, b3_ref, o_ref):
    nb, d = x_ref.shape
    n_params = w3_ref.shape[1]
    ic = _I_CHUNK

    xp = jnp.dot(x_ref[...], q_ref[...], preferred_element_type=jnp.float32)

    i_idx = jax.lax.broadcasted_iota(jnp.int32, (d, d), 0)
    j_idx = jax.lax.broadcasted_iota(jnp.int32, (d, d), 1)
    tri = (j_idx < i_idx).astype(jnp.float32)

    for c in range(d // ic):
        tri_c = tri[c * ic:(c + 1) * ic]
        xm = (tri_c[:, None, :] * xp[None, :, :]).reshape(ic * nb, d)
        xm = xm.astype(jnp.bfloat16)

        h = jnp.dot(xm, w1_ref[...], preferred_element_type=jnp.float32)
        h = jnp.maximum(h + b1_ref[...], 0.0).astype(jnp.bfloat16)
        h = jnp.dot(h, w2_ref[...], preferred_element_type=jnp.float32)
        h = jnp.maximum(h + b2_ref[...], 0.0).astype(jnp.bfloat16)
        h = (jnp.dot(h, w3_ref[...], preferred_element_type=jnp.float32)
             + b3_ref[...])

        for il in range(ic):
            i = c * ic + il
            o_ref[:, i * n_params:(i + 1) * n_params] = (
                h[il * nb:(il + 1) * nb, :])


@jax.jit
def _forward(x, Q, W1, b1, W2, b2, W3, b3):
    n, d = x.shape
    n_params = W3.shape[1]
    nb = _BLOCK_N

    xb = x.astype(jnp.bfloat16)
    qb = Q.astype(jnp.bfloat16)
    w1b = W1.astype(jnp.bfloat16)
    w2b = W2.astype(jnp.bfloat16)
    w3b = W3.astype(jnp.bfloat16)

    const = lambda i: (0, 0)

    out = pl.pallas_call(
        _fused_kernel,
        grid=(n // nb,),
        in_specs=[
            pl.BlockSpec((nb, d), lambda i: (i, 0)),
            pl.BlockSpec(qb.shape, const),
            pl.BlockSpec(w1b.shape, const),
            pl.BlockSpec(b1.shape, const),
            pl.BlockSpec(w2b.shape, const),
            pl.BlockSpec(b2.shape, const),
            pl.BlockSpec(w3b.shape, const),
            pl.BlockSpec(b3.shape, const),
        ],
        out_specs=pl.BlockSpec((nb, d * n_params), lambda i: (i, 0)),
        out_shape=jax.ShapeDtypeStruct((n, d * n_params), jnp.float32),
        compiler_params=pltpu.CompilerParams(
            dimension_semantics=("parallel",)),
    )(xb, qb, w1b, b1, w2b, b2, w3b, b3)

    return out


def kernel(x, Q, W1, b1, W2, b2, W3, b3):
    return _forward(x, Q, W1, b1, W2, b2, W3, b3)


# speedup vs baseline: 1.5303x; 1.0007x over previous
import jax
import jax.numpy as jnp
from jax.experimental import pallas as pl
from jax.experimental.pallas import tpu as pltpu

_BLOCK_N = 128
_I_CHUNK = 64


def _fused_kernel(x_ref, q_ref, w1_ref, b1_ref, w2_ref, b2_ref,
                  w3_ref, b3_ref, o_ref):
    nb, d = x_ref.shape
    n_params = w3_ref.shape[1]
    ic = _I_CHUNK

    xp = jnp.dot(x_ref[...], q_ref[...], preferred_element_type=jnp.float32)

    i_idx = jax.lax.broadcasted_iota(jnp.int32, (d, d), 0)
    j_idx = jax.lax.broadcasted_iota(jnp.int32, (d, d), 1)
    tri = (j_idx < i_idx).astype(jnp.float32)

    for c in range(d // ic):
        tri_c = tri[c * ic:(c + 1) * ic]
        xm = (tri_c[:, None, :] * xp[None, :, :]).reshape(ic * nb, d)
        xm = xm.astype(jnp.bfloat16)

        h = jnp.dot(xm, w1_ref[...], preferred_element_type=jnp.float32)
        h = jnp.maximum(h + b1_ref[...], 0.0).astype(jnp.bfloat16)
        h = jnp.dot(h, w2_ref[...], preferred_element_type=jnp.float32)
        h = jnp.maximum(h + b2_ref[...], 0.0).astype(jnp.bfloat16)
        h = (jnp.dot(h, w3_ref[...], preferred_element_type=jnp.float32)
             + b3_ref[...])

        for il in range(ic):
            i = c * ic + il
            o_ref[:, i * n_params:(i + 1) * n_params] = (
                h[il * nb:(il + 1) * nb, :])


@jax.jit
def _forward(x, Q, W1, b1, W2, b2, W3, b3):
    n, d = x.shape
    n_params = W3.shape[1]
    nb = _BLOCK_N

    xb = x.astype(jnp.bfloat16)
    qb = Q.astype(jnp.bfloat16)
    w1b = W1.astype(jnp.bfloat16)
    w2b = W2.astype(jnp.bfloat16)
    w3b = W3.astype(jnp.bfloat16)

    const = lambda i: (0, 0)

    out = pl.pallas_call(
        _fused_kernel,
        grid=(n // nb,),
        in_specs=[
            pl.BlockSpec((nb, d), lambda i: (i, 0)),
            pl.BlockSpec(qb.shape, const),
            pl.BlockSpec(w1b.shape, const),
            pl.BlockSpec(b1.shape, const),
            pl.BlockSpec(w2b.shape, const),
            pl.BlockSpec(b2.shape, const),
            pl.BlockSpec(w3b.shape, const),
            pl.BlockSpec(b3.shape, const),
        ],
        out_specs=pl.BlockSpec((nb, d * n_params), lambda i: (i, 0)),
        out_shape=jax.ShapeDtypeStruct((n, d * n_params), jnp.float32),
        compiler_params=pltpu.CompilerParams(
            dimension_semantics=("parallel",)),
    )(xb, qb, w1b, b1, w2b, b2, w3b, b3)

    return out


def kernel(x, Q, W1, b1, W2, b2, W3, b3):
    return _forward(x, Q, W1, b1, W2, b2, W3, b3)


# speedup vs baseline: 1.7107x; 1.1178x over previous
import jax
import jax.numpy as jnp
from jax.experimental import pallas as pl
from jax.experimental.pallas import tpu as pltpu

_BLOCK_N = 128
_I_CHUNK = 32


def _fused_kernel(x_ref, qt_ref, w1t_ref, b1t_ref, w2t_ref, b2t_ref,
                  w3t_ref, b3t_ref, o_ref):
    nb, d = x_ref.shape
    n_params = w3t_ref.shape[0]
    ic = _I_CHUNK

    xt = x_ref[...].T
    xpt = jnp.dot(qt_ref[...], xt, preferred_element_type=jnp.float32)

    for c in range(d // ic):
        j_io = jax.lax.broadcasted_iota(jnp.int32, (d, ic, nb), 0)
        i_io = jax.lax.broadcasted_iota(jnp.int32, (d, ic, nb), 1) + c * ic
        m = (j_io < i_io).astype(jnp.float32)
        xmt = (m * xpt[:, None, :]).reshape(d, ic * nb).astype(jnp.bfloat16)

        h = jnp.dot(w1t_ref[...], xmt, preferred_element_type=jnp.float32)
        h = jnp.maximum(h + b1t_ref[...], 0.0).astype(jnp.bfloat16)
        h = jnp.dot(w2t_ref[...], h, preferred_element_type=jnp.float32)
        h = jnp.maximum(h + b2t_ref[...], 0.0).astype(jnp.bfloat16)
        h = (jnp.dot(w3t_ref[...], h, preferred_element_type=jnp.float32)
             + b3t_ref[...])

        for il in range(ic):
            i = c * ic + il
            o_ref[:, i * n_params:(i + 1) * n_params] = (
                h[:, il * nb:(il + 1) * nb].T)


@jax.jit
def _forward(x, Q, W1, b1, W2, b2, W3, b3):
    n, d = x.shape
    n_params = W3.shape[1]
    nb = _BLOCK_N

    xb = x.astype(jnp.bfloat16)
    qt = Q.T.astype(jnp.bfloat16)
    w1t = W1.T.astype(jnp.bfloat16)
    w2t = W2.T.astype(jnp.bfloat16)
    w3t = W3.T.astype(jnp.bfloat16)
    b1t = b1.reshape(-1, 1)
    b2t = b2.reshape(-1, 1)
    b3t = b3.reshape(-1, 1)

    const = lambda i: (0, 0)

    out = pl.pallas_call(
        _fused_kernel,
        grid=(n // nb,),
        in_specs=[
            pl.BlockSpec((nb, d), lambda i: (i, 0)),
            pl.BlockSpec(qt.shape, const),
            pl.BlockSpec(w1t.shape, const),
            pl.BlockSpec(b1t.shape, const),
            pl.BlockSpec(w2t.shape, const),
            pl.BlockSpec(b2t.shape, const),
            pl.BlockSpec(w3t.shape, const),
            pl.BlockSpec(b3t.shape, const),
        ],
        out_specs=pl.BlockSpec((nb, d * n_params), lambda i: (i, 0)),
        out_shape=jax.ShapeDtypeStruct((n, d * n_params), jnp.float32),
        compiler_params=pltpu.CompilerParams(
            dimension_semantics=("parallel",)),
    )(xb, qt, w1t, b1t, w2t, b2t, w3t, b3t)

    return out


def kernel(x, Q, W1, b1, W2, b2, W3, b3):
    return _forward(x, Q, W1, b1, W2, b2, W3, b3)


# speedup vs baseline: 1.7108x; 1.0001x over previous
import jax
import jax.numpy as jnp
from jax.experimental import pallas as pl
from jax.experimental.pallas import tpu as pltpu

_BLOCK_N = 128
_I_CHUNK = 64


def _fused_kernel(x_ref, qt_ref, w1t_ref, b1t_ref, w2t_ref, b2t_ref,
                  w3t_ref, b3t_ref, o_ref):
    nb, d = x_ref.shape
    n_params = w3t_ref.shape[0]
    ic = _I_CHUNK

    xt = x_ref[...].T
    xpt = jnp.dot(qt_ref[...], xt, preferred_element_type=jnp.float32)

    for c in range(d // ic):
        j_io = jax.lax.broadcasted_iota(jnp.int32, (d, ic, nb), 0)
        i_io = jax.lax.broadcasted_iota(jnp.int32, (d, ic, nb), 1) + c * ic
        m = (j_io < i_io).astype(jnp.float32)
        xmt = (m * xpt[:, None, :]).reshape(d, ic * nb).astype(jnp.bfloat16)

        h = jnp.dot(w1t_ref[...], xmt, preferred_element_type=jnp.float32)
        h = jnp.maximum(h + b1t_ref[...], 0.0).astype(jnp.bfloat16)
        h = jnp.dot(w2t_ref[...], h, preferred_element_type=jnp.float32)
        h = jnp.maximum(h + b2t_ref[...], 0.0).astype(jnp.bfloat16)
        h = (jnp.dot(w3t_ref[...], h, preferred_element_type=jnp.float32)
             + b3t_ref[...])

        for il in range(ic):
            i = c * ic + il
            o_ref[:, i * n_params:(i + 1) * n_params] = (
                h[:, il * nb:(il + 1) * nb].T)


@jax.jit
def _forward(x, Q, W1, b1, W2, b2, W3, b3):
    n, d = x.shape
    n_params = W3.shape[1]
    nb = _BLOCK_N

    xb = x.astype(jnp.bfloat16)
    qt = Q.T.astype(jnp.bfloat16)
    w1t = W1.T.astype(jnp.bfloat16)
    w2t = W2.T.astype(jnp.bfloat16)
    w3t = W3.T.astype(jnp.bfloat16)
    b1t = b1.reshape(-1, 1)
    b2t = b2.reshape(-1, 1)
    b3t = b3.reshape(-1, 1)

    const = lambda i: (0, 0)

    out = pl.pallas_call(
        _fused_kernel,
        grid=(n // nb,),
        in_specs=[
            pl.BlockSpec((nb, d), lambda i: (i, 0)),
            pl.BlockSpec(qt.shape, const),
            pl.BlockSpec(w1t.shape, const),
            pl.BlockSpec(b1t.shape, const),
            pl.BlockSpec(w2t.shape, const),
            pl.BlockSpec(b2t.shape, const),
            pl.BlockSpec(w3t.shape, const),
            pl.BlockSpec(b3t.shape, const),
        ],
        out_specs=pl.BlockSpec((nb, d * n_params), lambda i: (i, 0)),
        out_shape=jax.ShapeDtypeStruct((n, d * n_params), jnp.float32),
        compiler_params=pltpu.CompilerParams(
            dimension_semantics=("parallel",)),
    )(xb, qt, w1t, b1t, w2t, b2t, w3t, b3t)

    return out


def kernel(x, Q, W1, b1, W2, b2, W3, b3):
    return _forward(x, Q, W1, b1, W2, b2, W3, b3)
